# Initial kernel scaffold; baseline (speedup 1.0000x reference)
import jax
import jax.numpy as jnp
from jax.experimental import pallas as pl


def kernel(x, edge_index, edge_weight, W1, b1, W2, b2, W3, b3, Wh1, bh1, Wh2, bh2):
    raise NotImplementedError("write your pallas kernel here")


# speedup vs baseline: 6.0170x; 6.0170x over previous
import functools

import jax
import jax.numpy as jnp
from jax import lax
from jax.experimental import pallas as pl
from jax.experimental.pallas import tpu as pltpu
from jax.experimental.pallas import tpu_sc as plsc

N = 10000
E = 160000
F = 256
H = 256
HH = 128
NPAD = 10240
BN = 1000
WIN = 128

_MESH = plsc.VectorSubcoreMesh(
    core_axis_name="c", subcore_axis_name="s", num_cores=2, num_subcores=16)

_f32 = jnp.float32
_i32 = jnp.int32


def _zero16(ref, num):
    def body(i, carry):
        ref[pl.ds(i * 16, 16)] = jnp.zeros((16,), _f32)
        return carry
    lax.fori_loop(0, num // 16, body, 0)


def _sc_deg(dst, ew):
    per_tile = E // 32
    nfull = per_tile // WIN
    rem = per_tile - nfull * WIN

    @functools.partial(
        pl.kernel,
        out_type=jax.ShapeDtypeStruct((2, NPAD), _f32),
        mesh=_MESH,
        compiler_params=pltpu.CompilerParams(needs_layout_passes=False),
        scratch_types=[
            pltpu.VMEM((WIN,), _i32),
            pltpu.VMEM((WIN,), _f32),
            pltpu.VMEM((rem,), _i32),
            pltpu.VMEM((rem,), _f32),
            pltpu.VMEM((640,), _f32),
            pltpu.VMEM_SHARED((NPAD,), _f32),
        ],
    )
    def deg_kernel(dst_hbm, ew_hbm, out_hbm, dst_v, ew_v, dst_r, ew_r,
                   zbuf, acc):
        c = lax.axis_index("c")
        s = lax.axis_index("s")
        _zero16(zbuf, 640)
        pltpu.sync_copy(zbuf, acc.at[pl.ds(s * 640, 640)])
        plsc.subcore_barrier()

        base0 = (c * 16 + s) * per_tile

        def win(w, carry):
            b = base0 + w * WIN
            pltpu.sync_copy(dst_hbm.at[pl.ds(b, WIN)], dst_v)
            pltpu.sync_copy(ew_hbm.at[pl.ds(b, WIN)], ew_v)
            pltpu.sync_copy(ew_v, acc.at[dst_v], add=True)
            return carry
        lax.fori_loop(0, nfull, win, 0)

        b = base0 + nfull * WIN
        pltpu.sync_copy(dst_hbm.at[pl.ds(b, rem)], dst_r)
        pltpu.sync_copy(ew_hbm.at[pl.ds(b, rem)], ew_r)
        pltpu.sync_copy(ew_r, acc.at[dst_r], add=True)

        plsc.subcore_barrier()
        pltpu.sync_copy(acc.at[pl.ds(s * 640, 640)],
                        out_hbm.at[c, pl.ds(s * 640, 640)])

    return deg_kernel(dst, ew)


def _sc_agg(xs_flat, src, dst, ew):
    per_tile = E // 16
    nfull = per_tile // WIN
    rem = per_tile - nfull * WIN

    @functools.partial(
        pl.kernel,
        out_type=jax.ShapeDtypeStruct((2 * NPAD, HH), _f32),
        mesh=_MESH,
        compiler_params=pltpu.CompilerParams(needs_layout_passes=False),
        scratch_types=[
            pltpu.VMEM((WIN,), _i32),
            pltpu.VMEM((WIN,), _i32),
            pltpu.VMEM((WIN,), _i32),
            pltpu.VMEM((WIN,), _f32),
            pltpu.VMEM((WIN, HH), _f32),
            pltpu.VMEM((rem,), _i32),
            pltpu.VMEM((rem,), _i32),
            pltpu.VMEM((rem,), _i32),
            pltpu.VMEM((rem,), _f32),
            pltpu.VMEM((rem, HH), _f32),
            pltpu.VMEM_SHARED((NPAD, HH), _f32),
            pltpu.SemaphoreType.DMA,
        ],
    )
    def agg_kernel(xs_hbm, src_hbm, dst_hbm, ew_hbm, out_hbm,
                   src_v, idx_v, dst_v, ew_v, rows,
                   src_r, idx_r, dst_r, ew_r, rows_r, acc, sem):
        c = lax.axis_index("c")
        s = lax.axis_index("s")
        off = c * N

        def zr(r, carry):
            for j in range(HH // 16):
                rows[r, pl.ds(j * 16, 16)] = jnp.zeros((16,), _f32)
            return carry
        lax.fori_loop(0, WIN, zr, 0)
        for k in range(5):
            pltpu.sync_copy(rows, acc.at[pl.ds(s * 640 + k * WIN, WIN)])
        plsc.subcore_barrier()

        base0 = s * per_tile

        def win(w, carry):
            b = base0 + w * WIN
            pltpu.sync_copy(src_hbm.at[pl.ds(b, WIN)], src_v)
            pltpu.sync_copy(dst_hbm.at[pl.ds(b, WIN)], dst_v)
            pltpu.sync_copy(ew_hbm.at[pl.ds(b, WIN)], ew_v)

            def adj(i, cc):
                idx_v[pl.ds(i * 16, 16)] = src_v[pl.ds(i * 16, 16)] + off
                return cc
            lax.fori_loop(0, WIN // 16, adj, 0)

            pltpu.async_copy(xs_hbm.at[idx_v], rows, sem).wait()

            def mul(e, cc):
                g = plsc.load_gather(ew_v, [jnp.full((16,), e, _i32)])
                for j in range(HH // 16):
                    rows[e, pl.ds(j * 16, 16)] = rows[e, pl.ds(j * 16, 16)] * g
                return cc
            lax.fori_loop(0, WIN, mul, 0)

            pltpu.sync_copy(rows, acc.at[dst_v], add=True)
            return carry
        lax.fori_loop(0, nfull, win, 0)

        b = base0 + nfull * WIN
        pltpu.sync_copy(src_hbm.at[pl.ds(b, rem)], src_r)
        pltpu.sync_copy(dst_hbm.at[pl.ds(b, rem)], dst_r)
        pltpu.sync_copy(ew_hbm.at[pl.ds(b, rem)], ew_r)
        idx_r[...] = src_r[...] + off
        pltpu.async_copy(xs_hbm.at[idx_r], rows_r, sem).wait()

        def mul_r(e, cc):
            g = plsc.load_gather(ew_r, [jnp.full((16,), e, _i32)])
            for j in range(HH // 16):
                rows_r[e, pl.ds(j * 16, 16)] = rows_r[e, pl.ds(j * 16, 16)] * g
            return cc
        lax.fori_loop(0, rem, mul_r, 0)
        pltpu.sync_copy(rows_r, acc.at[dst_r], add=True)

        plsc.subcore_barrier()
        pltpu.sync_copy(acc.at[pl.ds(s * 640, 640)],
                        out_hbm.at[pl.ds(c * NPAD + s * 640, 640)])

    return agg_kernel(xs_flat, src, dst, ew)


def _tc_prep1(x, W1, deg2):
    def body(x_ref, w_ref, deg_ref, xs_ref, dinv_ref):
        deg = deg_ref[0, :, 0] + deg_ref[1, :, 0] + 1.0
        dinv = lax.rsqrt(deg)
        xw = jnp.dot(x_ref[...], w_ref[...], preferred_element_type=_f32)
        xs = xw * dinv[:, None]
        xs_ref[0] = xs[:, :HH]
        xs_ref[1] = xs[:, HH:]
        dinv_ref[...] = dinv[:, None]

    return pl.pallas_call(
        body,
        grid=(N // BN,),
        in_specs=[
            pl.BlockSpec((BN, F), lambda i: (i, 0)),
            pl.BlockSpec((F, H), lambda i: (0, 0)),
            pl.BlockSpec((2, BN, 1), lambda i: (0, i, 0)),
        ],
        out_specs=[
            pl.BlockSpec((2, BN, HH), lambda i: (0, i, 0)),
            pl.BlockSpec((BN, 1), lambda i: (i, 0)),
        ],
        out_shape=[
            jax.ShapeDtypeStruct((2, N, HH), _f32),
            jax.ShapeDtypeStruct((N, 1), _f32),
        ],
    )(x, W1, deg2)


def _tc_mid(agg, xs_prev, dinv, b, Wn):
    def body(agg_ref, xsp_ref, dinv_ref, b_ref, w_ref, out_ref):
        pre = jnp.concatenate(
            [agg_ref[0] + xsp_ref[0], agg_ref[1] + xsp_ref[1]], axis=1)
        h = jax.nn.relu(pre * dinv_ref[...] + b_ref[...])
        xw = jnp.dot(h, w_ref[...], preferred_element_type=_f32)
        xs = xw * dinv_ref[...]
        out_ref[0] = xs[:, :HH]
        out_ref[1] = xs[:, HH:]

    return pl.pallas_call(
        body,
        grid=(N // BN,),
        in_specs=[
            pl.BlockSpec((2, BN, HH), lambda i: (0, i, 0)),
            pl.BlockSpec((2, BN, HH), lambda i: (0, i, 0)),
            pl.BlockSpec((BN, 1), lambda i: (i, 0)),
            pl.BlockSpec((1, H), lambda i: (0, 0)),
            pl.BlockSpec((H, H), lambda i: (0, 0)),
        ],
        out_specs=pl.BlockSpec((2, BN, HH), lambda i: (0, i, 0)),
        out_shape=jax.ShapeDtypeStruct((2, N, HH), _f32),
    )(agg, xs_prev, dinv, b, Wn)


def _tc_final(agg, xs_prev, dinv, b3, Wh1, bh1, Wh2, bh2):
    def body(agg_ref, xsp_ref, dinv_ref, b_ref, wh1_ref, bh1_ref, wh2_ref,
             bh2_ref, o0_ref, o1_ref, o2_ref, emb_ref):
        pre = jnp.concatenate(
            [agg_ref[0] + xsp_ref[0], agg_ref[1] + xsp_ref[1]], axis=1)
        emb = pre * dinv_ref[...] + b_ref[...]
        emb_ref[...] = emb
        outs = (o0_ref, o1_ref, o2_ref)
        for t in range(3):
            h = jax.nn.relu(
                jnp.dot(emb, wh1_ref[t], preferred_element_type=_f32)
                + bh1_ref[t])
            z = jnp.dot(h, wh2_ref[t], preferred_element_type=_f32)
            outs[t][...] = z + bh2_ref[t, 0]

    return pl.pallas_call(
        body,
        grid=(N // BN,),
        in_specs=[
            pl.BlockSpec((2, BN, HH), lambda i: (0, i, 0)),
            pl.BlockSpec((2, BN, HH), lambda i: (0, i, 0)),
            pl.BlockSpec((BN, 1), lambda i: (i, 0)),
            pl.BlockSpec((1, H), lambda i: (0, 0)),
            pl.BlockSpec((3, H, H), lambda i: (0, 0, 0)),
            pl.BlockSpec((3, 1, H), lambda i: (0, 0, 0)),
            pl.BlockSpec((3, H, 1), lambda i: (0, 0, 0)),
            pl.BlockSpec((3, 1), lambda i: (0, 0)),
        ],
        out_specs=[
            pl.BlockSpec((BN, 1), lambda i: (i, 0)),
            pl.BlockSpec((BN, 1), lambda i: (i, 0)),
            pl.BlockSpec((BN, 1), lambda i: (i, 0)),
            pl.BlockSpec((BN, H), lambda i: (i, 0)),
        ],
        out_shape=[
            jax.ShapeDtypeStruct((N, 1), _f32),
            jax.ShapeDtypeStruct((N, 1), _f32),
            jax.ShapeDtypeStruct((N, 1), _f32),
            jax.ShapeDtypeStruct((N, H), _f32),
        ],
    )(agg, xs_prev, dinv, b3, Wh1, bh1, Wh2, bh2)


def kernel(x, edge_index, edge_weight, W1, b1, W2, b2, W3, b3,
           Wh1, bh1, Wh2, bh2):
    src = edge_index[0]
    dst = edge_index[1]

    deg2 = _sc_deg(dst, edge_weight).reshape(2, NPAD, 1)
    xs1, dinv = _tc_prep1(x, W1, deg2)

    def layer_agg(xs):
        flat = xs.reshape(2 * N, HH)
        return _sc_agg(flat, src, dst, edge_weight).reshape(2, NPAD, HH)

    agg1 = layer_agg(xs1)
    xs2 = _tc_mid(agg1, xs1, dinv, b1.reshape(1, H), W2)
    agg2 = layer_agg(xs2)
    xs3 = _tc_mid(agg2, xs2, dinv, b2.reshape(1, H), W3)
    agg3 = layer_agg(xs3)
    o0, o1, o2, emb = _tc_final(agg3, xs3, dinv, b3.reshape(1, H),
                                Wh1, bh1.reshape(3, 1, H), Wh2, bh2)
    return (o0, o1, o2, emb)


# speedup vs baseline: 14.3527x; 2.3853x over previous
import functools

import jax
import jax.numpy as jnp
from jax import lax
from jax.experimental import pallas as pl
from jax.experimental.pallas import tpu as pltpu
from jax.experimental.pallas import tpu_sc as plsc

N = 10000
E = 160000
F = 256
H = 256
HH = 128
NPAD = 10240
BN = 1000
WIN = 128

_MESH = plsc.VectorSubcoreMesh(
    core_axis_name="c", subcore_axis_name="s", num_cores=2, num_subcores=16)

_f32 = jnp.float32
_i32 = jnp.int32


def _zero16(ref, num):
    def body(i, carry):
        ref[pl.ds(i * 16, 16)] = jnp.zeros((16,), _f32)
        return carry
    lax.fori_loop(0, num // 16, body, 0)


def _sc_deg(dst, ew):
    per_tile = E // 32
    nfull = per_tile // WIN
    rem = per_tile - nfull * WIN

    @functools.partial(
        pl.kernel,
        out_type=jax.ShapeDtypeStruct((2, NPAD), _f32),
        mesh=_MESH,
        compiler_params=pltpu.CompilerParams(needs_layout_passes=False),
        scratch_types=[
            pltpu.VMEM((WIN,), _i32),
            pltpu.VMEM((WIN,), _f32),
            pltpu.VMEM((rem,), _i32),
            pltpu.VMEM((rem,), _f32),
            pltpu.VMEM((640,), _f32),
            pltpu.VMEM_SHARED((NPAD,), _f32),
        ],
    )
    def deg_kernel(dst_hbm, ew_hbm, out_hbm, dst_v, ew_v, dst_r, ew_r,
                   zbuf, acc):
        c = lax.axis_index("c")
        s = lax.axis_index("s")
        _zero16(zbuf, 640)
        pltpu.sync_copy(zbuf, acc.at[pl.ds(s * 640, 640)])
        plsc.subcore_barrier()

        base0 = (c * 16 + s) * per_tile

        def win(w, carry):
            b = base0 + w * WIN
            pltpu.sync_copy(dst_hbm.at[pl.ds(b, WIN)], dst_v)
            pltpu.sync_copy(ew_hbm.at[pl.ds(b, WIN)], ew_v)
            pltpu.sync_copy(ew_v, acc.at[dst_v], add=True)
            return carry
        lax.fori_loop(0, nfull, win, 0)

        b = base0 + nfull * WIN
        pltpu.sync_copy(dst_hbm.at[pl.ds(b, rem)], dst_r)
        pltpu.sync_copy(ew_hbm.at[pl.ds(b, rem)], ew_r)
        pltpu.sync_copy(ew_r, acc.at[dst_r], add=True)

        plsc.subcore_barrier()
        pltpu.sync_copy(acc.at[pl.ds(s * 640, 640)],
                        out_hbm.at[c, pl.ds(s * 640, 640)])

    return deg_kernel(dst, ew)


_GDN = lax.GatherDimensionNumbers(
    offset_dims=(), collapsed_slice_dims=(0,), start_index_map=(0,))


def _splat(vec16, i):
    idx = jnp.full((16, 1), i, _i32)
    return lax.gather(vec16, idx, _GDN, (1,),
                      mode=lax.GatherScatterMode.PROMISE_IN_BOUNDS)


def _sc_agg(xs_flat, src, dst, ew):
    per_tile = E // 16
    nfull = per_tile // WIN
    rem = per_tile - nfull * WIN

    @functools.partial(
        pl.kernel,
        out_type=jax.ShapeDtypeStruct((2 * NPAD, HH), _f32),
        mesh=_MESH,
        compiler_params=pltpu.CompilerParams(needs_layout_passes=False),
        scratch_types=[
            pltpu.VMEM((WIN,), _i32),
            pltpu.VMEM((WIN,), _i32),
            pltpu.VMEM((WIN,), _i32),
            pltpu.VMEM((WIN,), _i32),
            pltpu.VMEM((WIN,), _i32),
            pltpu.VMEM((WIN,), _i32),
            pltpu.VMEM((WIN,), _f32),
            pltpu.VMEM((WIN,), _f32),
            pltpu.VMEM((WIN, HH), _f32),
            pltpu.VMEM((WIN, HH), _f32),
            pltpu.VMEM((rem,), _i32),
            pltpu.VMEM((rem,), _i32),
            pltpu.VMEM((rem,), _i32),
            pltpu.VMEM((rem,), _f32),
            pltpu.VMEM((rem, HH), _f32),
            pltpu.VMEM_SHARED((NPAD, HH), _f32),
        ] + [pltpu.SemaphoreType.DMA] * 8,
    )
    def agg_kernel(xs_hbm, src_hbm, dst_hbm, ew_hbm, out_hbm,
                   src0, src1, idx0, idx1, dst0, dst1, ew0, ew1,
                   rows0, rows1, src_r, idx_r, dst_r, ew_r, rows_r, acc,
                   si0, si1, sg0, sg1, ss0, ss1, sd0, sd1):
        srcs = (src0, src1)
        idxs = (idx0, idx1)
        dsts = (dst0, dst1)
        ews = (ew0, ew1)
        rows = (rows0, rows1)
        sem_i = (si0, si1)
        sem_g = (sg0, sg1)
        sem_s = (ss0, ss1)
        sem_d = (sd0, sd1)

        c = lax.axis_index("c")
        s = lax.axis_index("s")
        off = c * N
        base0 = s * per_tile
        nw = nfull

        def wbase(w):
            return base0 + jnp.minimum(w, nw - 1) * WIN

        def issue_srcew(w, b):
            bb = wbase(w)
            pltpu.async_copy(src_hbm.at[pl.ds(bb, WIN)], srcs[b], sem_i[b])
            pltpu.async_copy(ew_hbm.at[pl.ds(bb, WIN)], ews[b], sem_i[b])

        def wait_srcew(b):
            pltpu.make_async_copy(
                src_hbm.at[pl.ds(0, WIN)], srcs[b], sem_i[b]).wait()
            pltpu.make_async_copy(
                ew_hbm.at[pl.ds(0, WIN)], ews[b], sem_i[b]).wait()

        def issue_dst(w, b):
            pltpu.async_copy(dst_hbm.at[pl.ds(wbase(w), WIN)],
                             dsts[b], sem_d[b])

        def wait_dst(b):
            pltpu.make_async_copy(
                dst_hbm.at[pl.ds(0, WIN)], dsts[b], sem_d[b]).wait()

        def adjust(b):
            def adj(i, cc):
                idxs[b][pl.ds(i * 16, 16)] = (
                    srcs[b][pl.ds(i * 16, 16)] + off)
                return cc
            lax.fori_loop(0, WIN // 16, adj, 0)

        def issue_gather(b):
            pltpu.async_copy(xs_hbm.at[idxs[b]], rows[b], sem_g[b])

        def wait_gather(b):
            pltpu.make_async_copy(
                xs_hbm.at[idxs[b]], rows[b], sem_g[b]).wait()

        def mul(b):
            def mm(g16, cc):
                ew16 = ews[b][pl.ds(g16 * 16, 16)]
                for e16 in range(16):
                    e = g16 * 16 + e16
                    gg = _splat(ew16, e16)
                    for j in range(HH // 16):
                        rows[b][e, pl.ds(j * 16, 16)] = (
                            rows[b][e, pl.ds(j * 16, 16)] * gg)
                return cc
            lax.fori_loop(0, WIN // 16, mm, 0)

        def issue_scatter(b):
            pltpu.async_copy(rows[b], acc.at[dsts[b]], sem_s[b], add=True)

        def wait_scatter(b):
            pltpu.make_async_copy(rows[b], acc.at[dsts[b]], sem_s[b]).wait()

        def zr(r, carry):
            for j in range(HH // 16):
                rows0[r, pl.ds(j * 16, 16)] = jnp.zeros((16,), _f32)
            return carry
        lax.fori_loop(0, WIN, zr, 0)
        for k in range(5):
            pltpu.sync_copy(rows0, acc.at[pl.ds(s * 640 + k * WIN, WIN)])
        plsc.subcore_barrier()

        issue_srcew(0, 0)
        issue_dst(0, 0)
        wait_srcew(0)
        adjust(0)
        issue_gather(0)
        issue_srcew(1, 1)
        wait_srcew(1)
        adjust(1)
        issue_gather(1)
        issue_dst(1, 1)
        wait_gather(0)
        mul(0)
        wait_dst(0)
        issue_scatter(0)
        issue_srcew(2, 0)

        def body(w, bslot):
            nb = 1 - bslot
            wait_srcew(nb)
            adjust(nb)
            wait_scatter(nb)
            issue_gather(nb)
            issue_dst(w + 1, nb)
            wait_gather(bslot)
            mul(bslot)
            wait_dst(bslot)
            issue_scatter(bslot)
            issue_srcew(w + 2, bslot)

        def pair(p, cc):
            w = 2 * p + 1
            body(w, 1)
            body(w + 1, 0)
            return cc
        lax.fori_loop(0, (nw - 2) // 2, pair, 0)

        wait_srcew(0)
        wait_scatter(0)
        wait_gather(1)
        mul(1)
        wait_dst(1)
        issue_scatter(1)

        b = base0 + nfull * WIN
        pltpu.sync_copy(src_hbm.at[pl.ds(b, rem)], src_r)
        pltpu.sync_copy(dst_hbm.at[pl.ds(b, rem)], dst_r)
        pltpu.sync_copy(ew_hbm.at[pl.ds(b, rem)], ew_r)
        idx_r[...] = src_r[...] + off
        pltpu.async_copy(xs_hbm.at[idx_r], rows_r, sg0).wait()
        ew16r = ew_r[...]
        for e in range(rem):
            gg = _splat(ew16r, e)
            for j in range(HH // 16):
                rows_r[e, pl.ds(j * 16, 16)] = (
                    rows_r[e, pl.ds(j * 16, 16)] * gg)
        pltpu.sync_copy(rows_r, acc.at[dst_r], add=True)
        wait_scatter(1)

        plsc.subcore_barrier()
        pltpu.sync_copy(acc.at[pl.ds(s * 640, 640)],
                        out_hbm.at[pl.ds(c * NPAD + s * 640, 640)])

    return agg_kernel(xs_flat, src, dst, ew)


def _tc_prep1(x, W1, deg2):
    def body(x_ref, w_ref, deg_ref, xs_ref, dinv_ref):
        deg = deg_ref[0, :, 0] + deg_ref[1, :, 0] + 1.0
        dinv = lax.rsqrt(deg)
        xw = jnp.dot(x_ref[...], w_ref[...], preferred_element_type=_f32)
        xs = xw * dinv[:, None]
        xs_ref[0] = xs[:, :HH]
        xs_ref[1] = xs[:, HH:]
        dinv_ref[...] = dinv[:, None]

    return pl.pallas_call(
        body,
        grid=(N // BN,),
        in_specs=[
            pl.BlockSpec((BN, F), lambda i: (i, 0)),
            pl.BlockSpec((F, H), lambda i: (0, 0)),
            pl.BlockSpec((2, BN, 1), lambda i: (0, i, 0)),
        ],
        out_specs=[
            pl.BlockSpec((2, BN, HH), lambda i: (0, i, 0)),
            pl.BlockSpec((BN, 1), lambda i: (i, 0)),
        ],
        out_shape=[
            jax.ShapeDtypeStruct((2, N, HH), _f32),
            jax.ShapeDtypeStruct((N, 1), _f32),
        ],
    )(x, W1, deg2)


def _tc_mid(agg, xs_prev, dinv, b, Wn):
    def body(agg_ref, xsp_ref, dinv_ref, b_ref, w_ref, out_ref):
        pre = jnp.concatenate(
            [agg_ref[0] + xsp_ref[0], agg_ref[1] + xsp_ref[1]], axis=1)
        h = jax.nn.relu(pre * dinv_ref[...] + b_ref[...])
        xw = jnp.dot(h, w_ref[...], preferred_element_type=_f32)
        xs = xw * dinv_ref[...]
        out_ref[0] = xs[:, :HH]
        out_ref[1] = xs[:, HH:]

    return pl.pallas_call(
        body,
        grid=(N // BN,),
        in_specs=[
            pl.BlockSpec((2, BN, HH), lambda i: (0, i, 0)),
            pl.BlockSpec((2, BN, HH), lambda i: (0, i, 0)),
            pl.BlockSpec((BN, 1), lambda i: (i, 0)),
            pl.BlockSpec((1, H), lambda i: (0, 0)),
            pl.BlockSpec((H, H), lambda i: (0, 0)),
        ],
        out_specs=pl.BlockSpec((2, BN, HH), lambda i: (0, i, 0)),
        out_shape=jax.ShapeDtypeStruct((2, N, HH), _f32),
    )(agg, xs_prev, dinv, b, Wn)


def _tc_final(agg, xs_prev, dinv, b3, Wh1, bh1, Wh2, bh2):
    def body(agg_ref, xsp_ref, dinv_ref, b_ref, wh1_ref, bh1_ref, wh2_ref,
             bh2_ref, o0_ref, o1_ref, o2_ref, emb_ref):
        pre = jnp.concatenate(
            [agg_ref[0] + xsp_ref[0], agg_ref[1] + xsp_ref[1]], axis=1)
        emb = pre * dinv_ref[...] + b_ref[...]
        emb_ref[...] = emb
        outs = (o0_ref, o1_ref, o2_ref)
        for t in range(3):
            h = jax.nn.relu(
                jnp.dot(emb, wh1_ref[t], preferred_element_type=_f32)
                + bh1_ref[t])
            z = jnp.dot(h, wh2_ref[t], preferred_element_type=_f32)
            outs[t][...] = z + bh2_ref[t, 0]

    return pl.pallas_call(
        body,
        grid=(N // BN,),
        in_specs=[
            pl.BlockSpec((2, BN, HH), lambda i: (0, i, 0)),
            pl.BlockSpec((2, BN, HH), lambda i: (0, i, 0)),
            pl.BlockSpec((BN, 1), lambda i: (i, 0)),
            pl.BlockSpec((1, H), lambda i: (0, 0)),
            pl.BlockSpec((3, H, H), lambda i: (0, 0, 0)),
            pl.BlockSpec((3, 1, H), lambda i: (0, 0, 0)),
            pl.BlockSpec((3, H, 1), lambda i: (0, 0, 0)),
            pl.BlockSpec((3, 1), lambda i: (0, 0)),
        ],
        out_specs=[
            pl.BlockSpec((BN, 1), lambda i: (i, 0)),
            pl.BlockSpec((BN, 1), lambda i: (i, 0)),
            pl.BlockSpec((BN, 1), lambda i: (i, 0)),
            pl.BlockSpec((BN, H), lambda i: (i, 0)),
        ],
        out_shape=[
            jax.ShapeDtypeStruct((N, 1), _f32),
            jax.ShapeDtypeStruct((N, 1), _f32),
            jax.ShapeDtypeStruct((N, 1), _f32),
            jax.ShapeDtypeStruct((N, H), _f32),
        ],
    )(agg, xs_prev, dinv, b3, Wh1, bh1, Wh2, bh2)


def kernel(x, edge_index, edge_weight, W1, b1, W2, b2, W3, b3,
           Wh1, bh1, Wh2, bh2):
    src = edge_index[0]
    dst = edge_index[1]

    deg2 = _sc_deg(dst, edge_weight).reshape(2, NPAD, 1)
    xs1, dinv = _tc_prep1(x, W1, deg2)

    def layer_agg(xs):
        flat = xs.reshape(2 * N, HH)
        return _sc_agg(flat, src, dst, edge_weight).reshape(2, NPAD, HH)

    agg1 = layer_agg(xs1)
    xs2 = _tc_mid(agg1, xs1, dinv, b1.reshape(1, H), W2)
    agg2 = layer_agg(xs2)
    xs3 = _tc_mid(agg2, xs2, dinv, b2.reshape(1, H), W3)
    agg3 = layer_agg(xs3)
    o0, o1, o2, emb = _tc_final(agg3, xs3, dinv, b3.reshape(1, H),
                                Wh1, bh1.reshape(3, 1, H), Wh2, bh2)
    return (o0, o1, o2, emb)


# speedup vs baseline: 15.1082x; 1.0526x over previous
import functools

import jax
import jax.numpy as jnp
from jax import lax
from jax.experimental import pallas as pl
from jax.experimental.pallas import tpu as pltpu
from jax.experimental.pallas import tpu_sc as plsc

N = 10000
E = 160000
F = 256
H = 256
HH = 128
NPAD = 10240
BN = 1000
WIN = 128

_MESH = plsc.VectorSubcoreMesh(
    core_axis_name="c", subcore_axis_name="s", num_cores=2, num_subcores=16)

_f32 = jnp.float32
_i32 = jnp.int32


def _zero16(ref, num):
    def body(i, carry):
        ref[pl.ds(i * 16, 16)] = jnp.zeros((16,), _f32)
        return carry
    lax.fori_loop(0, num // 16, body, 0)


def _sc_deg(dst, ew):
    per_tile = E // 32
    nfull = per_tile // WIN
    rem = per_tile - nfull * WIN

    @functools.partial(
        pl.kernel,
        out_type=jax.ShapeDtypeStruct((2, NPAD), _f32),
        mesh=_MESH,
        compiler_params=pltpu.CompilerParams(needs_layout_passes=False),
        scratch_types=[
            pltpu.VMEM((WIN,), _i32),
            pltpu.VMEM((WIN,), _i32),
            pltpu.VMEM((WIN,), _f32),
            pltpu.VMEM((WIN,), _f32),
            pltpu.VMEM((rem,), _i32),
            pltpu.VMEM((rem,), _f32),
            pltpu.VMEM((640,), _f32),
            pltpu.VMEM_SHARED((NPAD,), _f32),
        ] + [pltpu.SemaphoreType.DMA] * 2,
    )
    def deg_kernel(dst_hbm, ew_hbm, out_hbm, dst0, dst1, ew0, ew1,
                   dst_r, ew_r, zbuf, acc, sm0, sm1):
        dsts = (dst0, dst1)
        ews = (ew0, ew1)
        sems = (sm0, sm1)
        c = lax.axis_index("c")
        s = lax.axis_index("s")
        _zero16(zbuf, 640)
        pltpu.sync_copy(zbuf, acc.at[pl.ds(s * 640, 640)])
        plsc.subcore_barrier()

        base0 = (c * 16 + s) * per_tile

        def issue(w, b):
            bb = base0 + jnp.minimum(w, nfull - 1) * WIN
            pltpu.async_copy(dst_hbm.at[pl.ds(bb, WIN)], dsts[b], sems[b])
            pltpu.async_copy(ew_hbm.at[pl.ds(bb, WIN)], ews[b], sems[b])

        def wait_in(b):
            pltpu.make_async_copy(
                dst_hbm.at[pl.ds(0, WIN)], dsts[b], sems[b]).wait()
            pltpu.make_async_copy(
                ew_hbm.at[pl.ds(0, WIN)], ews[b], sems[b]).wait()

        issue(0, 0)
        issue(1, 1)

        def pairw(p, carry):
            w = 2 * p
            for (wi, b) in ((w, 0), (w + 1, 1)):
                wait_in(b)
                pltpu.sync_copy(ews[b], acc.at[dsts[b]], add=True)
                issue(wi + 2, b)
            return carry
        lax.fori_loop(0, (nfull - 1) // 2, pairw, 0)

        wait_in(0)
        pltpu.sync_copy(ews[0], acc.at[dsts[0]], add=True)
        wait_in(1)

        b = base0 + nfull * WIN
        pltpu.sync_copy(dst_hbm.at[pl.ds(b, rem)], dst_r)
        pltpu.sync_copy(ew_hbm.at[pl.ds(b, rem)], ew_r)
        pltpu.sync_copy(ew_r, acc.at[dst_r], add=True)

        plsc.subcore_barrier()
        pltpu.sync_copy(acc.at[pl.ds(s * 640, 640)],
                        out_hbm.at[c, pl.ds(s * 640, 640)])

    return deg_kernel(dst, ew)


_GDN = lax.GatherDimensionNumbers(
    offset_dims=(), collapsed_slice_dims=(0,), start_index_map=(0,))


def _splat(vec16, i):
    idx = jnp.full((16, 1), i, _i32)
    return lax.gather(vec16, idx, _GDN, (1,),
                      mode=lax.GatherScatterMode.PROMISE_IN_BOUNDS)


def _sc_agg(xs_flat, src, dst, ew):
    per_tile = E // 16
    nfull = per_tile // WIN
    rem = per_tile - nfull * WIN

    @functools.partial(
        pl.kernel,
        out_type=jax.ShapeDtypeStruct((2 * NPAD, HH), _f32),
        mesh=_MESH,
        compiler_params=pltpu.CompilerParams(needs_layout_passes=False),
        scratch_types=[
            pltpu.VMEM((WIN,), _i32),
            pltpu.VMEM((WIN,), _i32),
            pltpu.VMEM((WIN,), _i32),
            pltpu.VMEM((WIN,), _i32),
            pltpu.VMEM((WIN,), _i32),
            pltpu.VMEM((WIN,), _i32),
            pltpu.VMEM((WIN,), _f32),
            pltpu.VMEM((WIN,), _f32),
            pltpu.VMEM((WIN, HH), _f32),
            pltpu.VMEM((WIN, HH), _f32),
            pltpu.VMEM((rem,), _i32),
            pltpu.VMEM((rem,), _i32),
            pltpu.VMEM((rem,), _i32),
            pltpu.VMEM((rem,), _f32),
            pltpu.VMEM((rem, HH), _f32),
            pltpu.VMEM_SHARED((NPAD, HH), _f32),
        ] + [pltpu.SemaphoreType.DMA] * 8,
    )
    def agg_kernel(xs_hbm, src_hbm, dst_hbm, ew_hbm, out_hbm,
                   src0, src1, idx0, idx1, dst0, dst1, ew0, ew1,
                   rows0, rows1, src_r, idx_r, dst_r, ew_r, rows_r, acc,
                   si0, si1, sg0, sg1, ss0, ss1, sd0, sd1):
        srcs = (src0, src1)
        idxs = (idx0, idx1)
        dsts = (dst0, dst1)
        ews = (ew0, ew1)
        rows = (rows0, rows1)
        sem_i = (si0, si1)
        sem_g = (sg0, sg1)
        sem_s = (ss0, ss1)
        sem_d = (sd0, sd1)

        c = lax.axis_index("c")
        s = lax.axis_index("s")
        off = c * N
        base0 = s * per_tile
        nw = nfull

        def wbase(w):
            return base0 + jnp.minimum(w, nw - 1) * WIN

        def issue_srcew(w, b):
            bb = wbase(w)
            pltpu.async_copy(src_hbm.at[pl.ds(bb, WIN)], srcs[b], sem_i[b])
            pltpu.async_copy(ew_hbm.at[pl.ds(bb, WIN)], ews[b], sem_i[b])

        def wait_srcew(b):
            pltpu.make_async_copy(
                src_hbm.at[pl.ds(0, WIN)], srcs[b], sem_i[b]).wait()
            pltpu.make_async_copy(
                ew_hbm.at[pl.ds(0, WIN)], ews[b], sem_i[b]).wait()

        def issue_dst(w, b):
            pltpu.async_copy(dst_hbm.at[pl.ds(wbase(w), WIN)],
                             dsts[b], sem_d[b])

        def wait_dst(b):
            pltpu.make_async_copy(
                dst_hbm.at[pl.ds(0, WIN)], dsts[b], sem_d[b]).wait()

        def adjust(b):
            def adj(i, cc):
                idxs[b][pl.ds(i * 16, 16)] = (
                    srcs[b][pl.ds(i * 16, 16)] + off)
                return cc
            lax.fori_loop(0, WIN // 16, adj, 0)

        def issue_gather(b):
            pltpu.async_copy(xs_hbm.at[idxs[b]], rows[b], sem_g[b])

        def wait_gather(b):
            pltpu.make_async_copy(
                xs_hbm.at[idxs[b]], rows[b], sem_g[b]).wait()

        def mul(b):
            def mm(g16, cc):
                ew16 = ews[b][pl.ds(g16 * 16, 16)]
                for e16 in range(16):
                    e = g16 * 16 + e16
                    gg = _splat(ew16, e16)
                    for j in range(HH // 16):
                        rows[b][e, pl.ds(j * 16, 16)] = (
                            rows[b][e, pl.ds(j * 16, 16)] * gg)
                return cc
            lax.fori_loop(0, WIN // 16, mm, 0)

        def issue_scatter(b):
            pltpu.async_copy(rows[b], acc.at[dsts[b]], sem_s[b], add=True)

        def wait_scatter(b):
            pltpu.make_async_copy(rows[b], acc.at[dsts[b]], sem_s[b]).wait()

        def zr(r, carry):
            for j in range(HH // 16):
                rows0[r, pl.ds(j * 16, 16)] = jnp.zeros((16,), _f32)
            return carry
        lax.fori_loop(0, WIN, zr, 0)
        for k in range(5):
            pltpu.sync_copy(rows0, acc.at[pl.ds(s * 640 + k * WIN, WIN)])
        plsc.subcore_barrier()

        issue_srcew(0, 0)
        issue_dst(0, 0)
        wait_srcew(0)
        adjust(0)
        issue_gather(0)
        issue_srcew(1, 1)
        wait_srcew(1)
        adjust(1)
        issue_gather(1)
        issue_dst(1, 1)
        wait_gather(0)
        mul(0)
        wait_dst(0)
        issue_scatter(0)
        issue_srcew(2, 0)

        def body(w, bslot):
            nb = 1 - bslot
            wait_srcew(nb)
            adjust(nb)
            wait_scatter(nb)
            issue_gather(nb)
            issue_dst(w + 1, nb)
            wait_gather(bslot)
            mul(bslot)
            wait_dst(bslot)
            issue_scatter(bslot)
            issue_srcew(w + 2, bslot)

        def pair(p, cc):
            w = 2 * p + 1
            body(w, 1)
            body(w + 1, 0)
            return cc
        lax.fori_loop(0, (nw - 2) // 2, pair, 0)

        wait_srcew(0)
        wait_scatter(0)
        wait_gather(1)
        mul(1)
        wait_dst(1)
        issue_scatter(1)

        b = base0 + nfull * WIN
        pltpu.sync_copy(src_hbm.at[pl.ds(b, rem)], src_r)
        pltpu.sync_copy(dst_hbm.at[pl.ds(b, rem)], dst_r)
        pltpu.sync_copy(ew_hbm.at[pl.ds(b, rem)], ew_r)
        idx_r[...] = src_r[...] + off
        pltpu.async_copy(xs_hbm.at[idx_r], rows_r, sg0).wait()
        ew16r = ew_r[...]
        for e in range(rem):
            gg = _splat(ew16r, e)
            for j in range(HH // 16):
                rows_r[e, pl.ds(j * 16, 16)] = (
                    rows_r[e, pl.ds(j * 16, 16)] * gg)
        pltpu.sync_copy(rows_r, acc.at[dst_r], add=True)
        wait_scatter(1)

        plsc.subcore_barrier()
        pltpu.sync_copy(acc.at[pl.ds(s * 640, 640)],
                        out_hbm.at[pl.ds(c * NPAD + s * 640, 640)])

    return agg_kernel(xs_flat, src, dst, ew)


def _tc_xw1(x, W1):
    def body(x_ref, w_ref, xw_ref):
        xw = jnp.dot(x_ref[...], w_ref[...], preferred_element_type=_f32)
        xw_ref[0] = xw[:, :HH]
        xw_ref[1] = xw[:, HH:]

    return pl.pallas_call(
        body,
        grid=(N // BN,),
        in_specs=[
            pl.BlockSpec((BN, F), lambda i: (i, 0)),
            pl.BlockSpec((F, H), lambda i: (0, 0)),
        ],
        out_specs=pl.BlockSpec((2, BN, HH), lambda i: (0, i, 0)),
        out_shape=jax.ShapeDtypeStruct((2, N, HH), _f32),
    )(x, W1)


def _tc_scale1(xw, deg2):
    def body(xw_ref, deg_ref, xs_ref, dinv_ref):
        deg = deg_ref[0, :, 0] + deg_ref[1, :, 0] + 1.0
        dinv = lax.rsqrt(deg)
        xs_ref[0] = xw_ref[0] * dinv[:, None]
        xs_ref[1] = xw_ref[1] * dinv[:, None]
        dinv_ref[...] = dinv[:, None]

    return pl.pallas_call(
        body,
        grid=(N // BN,),
        in_specs=[
            pl.BlockSpec((2, BN, HH), lambda i: (0, i, 0)),
            pl.BlockSpec((2, BN, 1), lambda i: (0, i, 0)),
        ],
        out_specs=[
            pl.BlockSpec((2, BN, HH), lambda i: (0, i, 0)),
            pl.BlockSpec((BN, 1), lambda i: (i, 0)),
        ],
        out_shape=[
            jax.ShapeDtypeStruct((2, N, HH), _f32),
            jax.ShapeDtypeStruct((N, 1), _f32),
        ],
    )(xw, deg2)


def _tc_mid(agg, xs_prev, dinv, b, Wn):
    def body(agg_ref, xsp_ref, dinv_ref, b_ref, w_ref, out_ref):
        pre = jnp.concatenate(
            [agg_ref[0] + xsp_ref[0], agg_ref[1] + xsp_ref[1]], axis=1)
        h = jax.nn.relu(pre * dinv_ref[...] + b_ref[...])
        xw = jnp.dot(h, w_ref[...], preferred_element_type=_f32)
        xs = xw * dinv_ref[...]
        out_ref[0] = xs[:, :HH]
        out_ref[1] = xs[:, HH:]

    return pl.pallas_call(
        body,
        grid=(N // BN,),
        in_specs=[
            pl.BlockSpec((2, BN, HH), lambda i: (0, i, 0)),
            pl.BlockSpec((2, BN, HH), lambda i: (0, i, 0)),
            pl.BlockSpec((BN, 1), lambda i: (i, 0)),
            pl.BlockSpec((1, H), lambda i: (0, 0)),
            pl.BlockSpec((H, H), lambda i: (0, 0)),
        ],
        out_specs=pl.BlockSpec((2, BN, HH), lambda i: (0, i, 0)),
        out_shape=jax.ShapeDtypeStruct((2, N, HH), _f32),
    )(agg, xs_prev, dinv, b, Wn)


def _tc_final(agg, xs_prev, dinv, b3, Wh1, bh1, Wh2, bh2):
    def body(agg_ref, xsp_ref, dinv_ref, b_ref, wh1_ref, bh1_ref, wh2_ref,
             bh2_ref, o0_ref, o1_ref, o2_ref, emb_ref):
        pre = jnp.concatenate(
            [agg_ref[0] + xsp_ref[0], agg_ref[1] + xsp_ref[1]], axis=1)
        emb = pre * dinv_ref[...] + b_ref[...]
        emb_ref[...] = emb
        outs = (o0_ref, o1_ref, o2_ref)
        for t in range(3):
            h = jax.nn.relu(
                jnp.dot(emb, wh1_ref[t], preferred_element_type=_f32)
                + bh1_ref[t])
            z = jnp.dot(h, wh2_ref[t], preferred_element_type=_f32)
            outs[t][...] = z + bh2_ref[t, 0]

    return pl.pallas_call(
        body,
        grid=(N // BN,),
        in_specs=[
            pl.BlockSpec((2, BN, HH), lambda i: (0, i, 0)),
            pl.BlockSpec((2, BN, HH), lambda i: (0, i, 0)),
            pl.BlockSpec((BN, 1), lambda i: (i, 0)),
            pl.BlockSpec((1, H), lambda i: (0, 0)),
            pl.BlockSpec((3, H, H), lambda i: (0, 0, 0)),
            pl.BlockSpec((3, 1, H), lambda i: (0, 0, 0)),
            pl.BlockSpec((3, H, 1), lambda i: (0, 0, 0)),
            pl.BlockSpec((3, 1), lambda i: (0, 0)),
        ],
        out_specs=[
            pl.BlockSpec((BN, 1), lambda i: (i, 0)),
            pl.BlockSpec((BN, 1), lambda i: (i, 0)),
            pl.BlockSpec((BN, 1), lambda i: (i, 0)),
            pl.BlockSpec((BN, H), lambda i: (i, 0)),
        ],
        out_shape=[
            jax.ShapeDtypeStruct((N, 1), _f32),
            jax.ShapeDtypeStruct((N, 1), _f32),
            jax.ShapeDtypeStruct((N, 1), _f32),
            jax.ShapeDtypeStruct((N, H), _f32),
        ],
    )(agg, xs_prev, dinv, b3, Wh1, bh1, Wh2, bh2)


def kernel(x, edge_index, edge_weight, W1, b1, W2, b2, W3, b3,
           Wh1, bh1, Wh2, bh2):
    src = edge_index[0]
    dst = edge_index[1]

    deg2 = _sc_deg(dst, edge_weight).reshape(2, NPAD, 1)
    xw1 = _tc_xw1(x, W1)
    xs1, dinv = _tc_scale1(xw1, deg2)

    def layer_agg(xs):
        flat = xs.reshape(2 * N, HH)
        return _sc_agg(flat, src, dst, edge_weight).reshape(2, NPAD, HH)

    agg1 = layer_agg(xs1)
    xs2 = _tc_mid(agg1, xs1, dinv, b1.reshape(1, H), W2)
    agg2 = layer_agg(xs2)
    xs3 = _tc_mid(agg2, xs2, dinv, b2.reshape(1, H), W3)
    agg3 = layer_agg(xs3)
    o0, o1, o2, emb = _tc_final(agg3, xs3, dinv, b3.reshape(1, H),
                                Wh1, bh1.reshape(3, 1, H), Wh2, bh2)
    return (o0, o1, o2, emb)
